# Initial kernel scaffold; baseline (speedup 1.0000x reference)
import functools

import jax
import jax.numpy as jnp
from jax import lax
from jax.experimental import pallas as pl
from jax.experimental.pallas import tpu as pltpu
from jax.experimental.pallas import tpu_sc as plsc

N = 10000
E = 320000
HID = 128
NCLS = 8
L = 4

NPAD = 10240
NW = 32
EW = E // NW
K = 125
NCH = EW // K
ROWS_PER_SUB = NPAD // 16

_MESH = plsc.VectorSubcoreMesh(core_axis_name="c", subcore_axis_name="s")


@functools.partial(
    pl.kernel,
    out_type=(
        jax.ShapeDtypeStruct((NPAD, HID), jnp.float32),
        jax.ShapeDtypeStruct((2, NPAD, 16), jnp.float32),
    ),
    mesh=_MESH,
    scratch_types=[
        pltpu.VMEM((4, 80), jnp.int32),
        pltpu.VMEM((80, HID), jnp.float32),
        pltpu.VMEM((NCH, K), jnp.int32),
        pltpu.VMEM((K, 16), jnp.float32),
        pltpu.VMEM_SHARED((NPAD, 16), jnp.float32),
        pltpu.SemaphoreType.DMA,
    ],
)
def _sc_embed_count(emb_hbm, hidx_hbm, dstg_hbm, ones_hbm, z16_hbm,
                    x0_hbm, cntp_hbm,
                    hidx_v, rows_v, dst_v, ones_v, cntacc, sem):
    c = lax.axis_index("c")
    s = lax.axis_index("s")
    w = c * 16 + s

    pltpu.sync_copy(hidx_hbm.at[w], hidx_v)
    for jj in range(4):
        pltpu.async_copy(emb_hbm.at[hidx_v.at[jj]], rows_v, sem).wait()
        pltpu.sync_copy(rows_v, x0_hbm.at[pl.ds(w * 320 + jj * 80, 80)])

    pltpu.sync_copy(z16_hbm.at[pl.ds(s * ROWS_PER_SUB, ROWS_PER_SUB)],
                    cntacc.at[pl.ds(s * ROWS_PER_SUB, ROWS_PER_SUB)])
    pltpu.sync_copy(ones_hbm, ones_v)
    pltpu.sync_copy(dstg_hbm.at[w], dst_v)
    plsc.subcore_barrier()

    def body(j, _):
        pltpu.sync_copy(ones_v, cntacc.at[dst_v.at[j]], add=True)
        return ()
    lax.fori_loop(0, NCH, body, ())

    plsc.subcore_barrier()
    pltpu.sync_copy(cntacc.at[pl.ds(s * ROWS_PER_SUB, ROWS_PER_SUB)],
                    cntp_hbm.at[c, pl.ds(s * ROWS_PER_SUB, ROWS_PER_SUB)])


@functools.partial(
    pl.kernel,
    out_type=jax.ShapeDtypeStruct((2, NPAD, HID), jnp.float32),
    mesh=_MESH,
    scratch_types=[
        pltpu.VMEM((NCH, K), jnp.int32),
        pltpu.VMEM((NCH, K), jnp.int32),
        pltpu.VMEM((K, HID), jnp.float32),
        pltpu.VMEM((K, HID), jnp.float32),
        pltpu.VMEM_SHARED((NPAD, HID), jnp.float32),
        pltpu.SemaphoreType.DMA,
    ],
)
def _sc_segment_sum(x_hbm, srcg_hbm, dstg_hbm, z128_hbm,
                    parts_hbm,
                    src_v, dst_v, rows0, rows1, acc, sem):
    c = lax.axis_index("c")
    s = lax.axis_index("s")
    w = c * 16 + s

    pltpu.sync_copy(z128_hbm.at[pl.ds(s * ROWS_PER_SUB, ROWS_PER_SUB)],
                    acc.at[pl.ds(s * ROWS_PER_SUB, ROWS_PER_SUB)])
    pltpu.sync_copy(srcg_hbm.at[w], src_v)
    pltpu.sync_copy(dstg_hbm.at[w], dst_v)
    plsc.subcore_barrier()

    def wait_rows(buf):
        pltpu.make_async_copy(x_hbm.at[src_v.at[0]], buf, sem).wait()

    pltpu.async_copy(x_hbm.at[src_v.at[0]], rows0, sem)

    def body(i, _):
        j0 = 2 * i
        j1 = j0 + 1
        wait_rows(rows0)
        pltpu.async_copy(x_hbm.at[src_v.at[j1]], rows1, sem)
        pltpu.sync_copy(rows0, acc.at[dst_v.at[j0]], add=True)
        wait_rows(rows1)

        @pl.when(j1 + 1 < NCH)
        def _():
            pltpu.async_copy(x_hbm.at[src_v.at[j1 + 1]], rows0, sem)

        pltpu.sync_copy(rows1, acc.at[dst_v.at[j1]], add=True)
        return ()

    lax.fori_loop(0, NCH // 2, body, ())

    plsc.subcore_barrier()
    pltpu.sync_copy(acc.at[pl.ds(s * ROWS_PER_SUB, ROWS_PER_SUB)],
                    parts_hbm.at[c, pl.ds(s * ROWS_PER_SUB, ROWS_PER_SUB)])


def _tc_layer_body(x_ref, parts_ref, cntp_ref, w_ref, b_ref, g_ref, be_ref,
                   o_ref, *, mlp_refs=None):
    x = x_ref[...]
    agg = parts_ref[0] + parts_ref[1]
    cnt = cntp_ref[0, :, 0:1] + cntp_ref[1, :, 0:1]
    neigh = agg / jnp.maximum(cnt, 1.0)
    w_self = w_ref[:HID, :]
    w_neigh = w_ref[HID:, :]
    bundle = (
        jnp.dot(x, w_self, preferred_element_type=jnp.float32)
        + jnp.dot(neigh, w_neigh, preferred_element_type=jnp.float32)
        + b_ref[...]
    )
    ss = jnp.sum(bundle * bundle, axis=1, keepdims=True)
    bundle = bundle / jnp.maximum(jnp.sqrt(ss), 1e-12)
    bundle = jnp.maximum(bundle, 0.0)
    real = bundle[:N, :]
    mean = jnp.sum(real, axis=0, keepdims=True) * (1.0 / N)
    var = jnp.sum((real - mean) * (real - mean), axis=0, keepdims=True) * (1.0 / N)
    bundle = (bundle - mean) * lax.rsqrt(var + 1e-5) * g_ref[...] + be_ref[...]
    out = x + bundle
    if mlp_refs is None:
        o_ref[...] = out
    else:
        (m1, c1, m2, c2, m3, c3) = mlp_refs
        y = jnp.dot(out, m1[...], preferred_element_type=jnp.float32) + c1[...]
        y = jnp.maximum(y, 0.0)
        y = jnp.dot(y, m2[...], preferred_element_type=jnp.float32) + c2[...]
        y = jnp.maximum(y, 0.0)
        y = jnp.dot(y, m3[...], preferred_element_type=jnp.float32) + c3[...]
        o_ref[...] = y


def _tc_layer(x, parts, cntp, w, b, g, be):
    return pl.pallas_call(
        _tc_layer_body,
        out_shape=jax.ShapeDtypeStruct((NPAD, HID), jnp.float32),
    )(x, parts, cntp, w, b, g, be)


def _tc_layer_mlp(x, parts, cntp, w, b, g, be, m1, c1, m2, c2, m3, c3):
    def body(x_ref, parts_ref, cntp_ref, w_ref, b_ref, g_ref, be_ref,
             m1r, c1r, m2r, c2r, m3r, c3r, o_ref):
        _tc_layer_body(x_ref, parts_ref, cntp_ref, w_ref, b_ref, g_ref,
                       be_ref, o_ref,
                       mlp_refs=(m1r, c1r, m2r, c2r, m3r, c3r))
    return pl.pallas_call(
        body,
        out_shape=jax.ShapeDtypeStruct((NPAD, HID), jnp.float32),
    )(x, parts, cntp, w, b, g, be, m1, c1, m2, c2, m3, c3)


def kernel(h, edge_index, e, emb, Ws, bs, gammas, betas, mWs, mbs):
    del e
    f32 = jnp.float32

    h_pad = jnp.concatenate([h, jnp.zeros((NPAD - N,), jnp.int32)])
    hidx = h_pad.reshape(NW, 4, 80)
    srcg = edge_index[0].reshape(NW, NCH, K)
    dstg = edge_index[1].reshape(NW, NCH, K)
    ones_rows = jnp.ones((K, 16), f32)
    z16 = jnp.zeros((NPAD, 16), f32)
    z128 = jnp.zeros((NPAD, HID), f32)

    def pad_mat(m, rows, cols):
        return jnp.zeros((rows, cols), f32).at[: m.shape[0], : m.shape[1]].set(m)

    m1 = pad_mat(mWs[0], HID, HID)
    m2 = pad_mat(mWs[1], HID, HID)
    m3 = pad_mat(mWs[2], HID, HID)
    c1 = pad_mat(mbs[0][None, :], 1, HID)
    c2 = pad_mat(mbs[1][None, :], 1, HID)
    c3 = pad_mat(mbs[2][None, :], 1, HID)

    x, cntp = _sc_embed_count(emb, hidx, dstg, ones_rows, z16)

    for l in range(L):
        parts = _sc_segment_sum(x, srcg, dstg, z128)
        b2 = bs[l][None, :]
        g2 = gammas[l][None, :]
        be2 = betas[l][None, :]
        if l < L - 1:
            x = _tc_layer(x, parts, cntp, Ws[l], b2, g2, be2)
        else:
            x = _tc_layer_mlp(x, parts, cntp, Ws[l], b2, g2, be2,
                              m1, c1, m2, c2, m3, c3)

    return x[:N, :NCLS]


# speedup vs baseline: 2.8460x; 2.8460x over previous
import functools

import jax
import jax.numpy as jnp
from jax import lax
from jax.experimental import pallas as pl
from jax.experimental.pallas import tpu as pltpu
from jax.experimental.pallas import tpu_sc as plsc

N = 10000
E = 320000
HID = 128
NCLS = 8
L = 4

NPAD = 10240
NW = 32
K = 128
NCH = 80
GSZ = 16
NG = NCH // GSZ
EPAD = NW * NCH * K
ROWS_PER_SUB = NPAD // 16


def _sc_embed_count_body(emb_hbm, hidx_hbm, dstg_hbm, ones_hbm, z128_hbm,
                    x0_hbm, cntp_hbm,
                    hidx_v, rows_v, dst_v, ones_v, cntacc, sem):
    c = lax.axis_index("c")
    s = lax.axis_index("s")
    w = c * 16 + s

    pltpu.sync_copy(hidx_hbm.at[w], hidx_v)
    for jj in range(4):
        pltpu.async_copy(emb_hbm.at[hidx_v.at[jj]], rows_v, sem).wait()
        pltpu.sync_copy(rows_v, x0_hbm.at[pl.ds(w * 320 + jj * 80, 80)])

    pltpu.sync_copy(z128_hbm.at[pl.ds(s * ROWS_PER_SUB, ROWS_PER_SUB)],
                    cntacc.at[pl.ds(s * ROWS_PER_SUB, ROWS_PER_SUB)])
    pltpu.sync_copy(ones_hbm, ones_v)
    plsc.subcore_barrier()

    for g in range(NG):
        pltpu.sync_copy(dstg_hbm.at[w, pl.ds(g * GSZ, GSZ)], dst_v)

        def body(j, _):
            pltpu.sync_copy(ones_v, cntacc.at[dst_v.at[j]], add=True)
            return ()
        lax.fori_loop(0, GSZ, body, ())

    plsc.subcore_barrier()
    pltpu.sync_copy(cntacc.at[pl.ds(s * ROWS_PER_SUB, ROWS_PER_SUB)],
                    cntp_hbm.at[c, pl.ds(s * ROWS_PER_SUB, ROWS_PER_SUB)])


def _sc_segment_sum_body(x_hbm, srcg_hbm, dstg_hbm, z128_hbm,
                    parts_hbm,
                    src_v, dst_v, rows0, rows1, acc, sem):
    c = lax.axis_index("c")
    s = lax.axis_index("s")
    w = c * 16 + s

    pltpu.sync_copy(z128_hbm.at[pl.ds(s * ROWS_PER_SUB, ROWS_PER_SUB)],
                    acc.at[pl.ds(s * ROWS_PER_SUB, ROWS_PER_SUB)])
    plsc.subcore_barrier()

    for g in range(NG):
        pltpu.sync_copy(srcg_hbm.at[w, pl.ds(g * GSZ, GSZ)], src_v)
        pltpu.sync_copy(dstg_hbm.at[w, pl.ds(g * GSZ, GSZ)], dst_v)
        pltpu.async_copy(x_hbm.at[src_v.at[0]], rows0, sem)
        for i in range(GSZ // 2):
            j0, j1 = 2 * i, 2 * i + 1
            pltpu.make_async_copy(x_hbm.at[src_v.at[j0]], rows0, sem).wait()
            pltpu.async_copy(x_hbm.at[src_v.at[j1]], rows1, sem)
            pltpu.sync_copy(rows0, acc.at[dst_v.at[j0]], add=True)
            pltpu.make_async_copy(x_hbm.at[src_v.at[j1]], rows1, sem).wait()
            if j1 + 1 < GSZ:
                pltpu.async_copy(x_hbm.at[src_v.at[j1 + 1]], rows0, sem)
            pltpu.sync_copy(rows1, acc.at[dst_v.at[j1]], add=True)

    plsc.subcore_barrier()
    pltpu.sync_copy(acc.at[pl.ds(s * ROWS_PER_SUB, ROWS_PER_SUB)],
                    parts_hbm.at[c, pl.ds(s * ROWS_PER_SUB, ROWS_PER_SUB)])


@functools.cache
def _sc_kernels():
    mesh = plsc.VectorSubcoreMesh(core_axis_name="c", subcore_axis_name="s")
    embed = pl.kernel(
        _sc_embed_count_body,
        out_type=(
            jax.ShapeDtypeStruct((NPAD, HID), jnp.float32),
            jax.ShapeDtypeStruct((2, NPAD, HID), jnp.float32),
        ),
        mesh=mesh,
        scratch_types=[
            pltpu.VMEM((4, 80), jnp.int32),
            pltpu.VMEM((80, HID), jnp.float32),
            pltpu.VMEM((GSZ, K), jnp.int32),
            pltpu.VMEM((K, HID), jnp.float32),
            pltpu.VMEM_SHARED((NPAD, HID), jnp.float32),
            pltpu.SemaphoreType.DMA,
        ],
    )
    seg = pl.kernel(
        _sc_segment_sum_body,
        out_type=jax.ShapeDtypeStruct((2, NPAD, HID), jnp.float32),
        mesh=mesh,
        scratch_types=[
            pltpu.VMEM((GSZ, K), jnp.int32),
            pltpu.VMEM((GSZ, K), jnp.int32),
            pltpu.VMEM((K, HID), jnp.float32),
            pltpu.VMEM((K, HID), jnp.float32),
            pltpu.VMEM_SHARED((NPAD, HID), jnp.float32),
            pltpu.SemaphoreType.DMA,
        ],
    )
    return embed, seg


def _tc_layer_body(x_ref, parts_ref, cntp_ref, w_ref, b_ref, g_ref, be_ref,
                   o_ref, *, mlp_refs=None):
    x = x_ref[...]
    agg = parts_ref[0] + parts_ref[1]
    cnt = cntp_ref[0] + cntp_ref[1]
    neigh = agg / jnp.maximum(cnt, 1.0)
    w_self = w_ref[:HID, :]
    w_neigh = w_ref[HID:, :]
    bundle = (
        jnp.dot(x, w_self, preferred_element_type=jnp.float32)
        + jnp.dot(neigh, w_neigh, preferred_element_type=jnp.float32)
        + b_ref[...]
    )
    ss = jnp.sum(bundle * bundle, axis=1, keepdims=True)
    bundle = bundle / jnp.maximum(jnp.sqrt(ss), 1e-12)
    bundle = jnp.maximum(bundle, 0.0)
    real = bundle[:N, :]
    mean = jnp.sum(real, axis=0, keepdims=True) * (1.0 / N)
    var = jnp.sum((real - mean) * (real - mean), axis=0, keepdims=True) * (1.0 / N)
    bundle = (bundle - mean) * lax.rsqrt(var + 1e-5) * g_ref[...] + be_ref[...]
    out = x + bundle
    if mlp_refs is None:
        o_ref[...] = out
    else:
        (m1, c1, m2, c2, m3, c3) = mlp_refs
        y = jnp.dot(out, m1[...], preferred_element_type=jnp.float32) + c1[...]
        y = jnp.maximum(y, 0.0)
        y = jnp.dot(y, m2[...], preferred_element_type=jnp.float32) + c2[...]
        y = jnp.maximum(y, 0.0)
        y = jnp.dot(y, m3[...], preferred_element_type=jnp.float32) + c3[...]
        o_ref[...] = y


def _tc_layer(x, parts, cntp, w, b, g, be):
    return pl.pallas_call(
        _tc_layer_body,
        out_shape=jax.ShapeDtypeStruct((NPAD, HID), jnp.float32),
    )(x, parts, cntp, w, b, g, be)


def _tc_layer_mlp(x, parts, cntp, w, b, g, be, m1, c1, m2, c2, m3, c3):
    def body(x_ref, parts_ref, cntp_ref, w_ref, b_ref, g_ref, be_ref,
             m1r, c1r, m2r, c2r, m3r, c3r, o_ref):
        _tc_layer_body(x_ref, parts_ref, cntp_ref, w_ref, b_ref, g_ref,
                       be_ref, o_ref,
                       mlp_refs=(m1r, c1r, m2r, c2r, m3r, c3r))
    return pl.pallas_call(
        body,
        out_shape=jax.ShapeDtypeStruct((NPAD, HID), jnp.float32),
    )(x, parts, cntp, w, b, g, be, m1, c1, m2, c2, m3, c3)


def kernel(h, edge_index, e, emb, Ws, bs, gammas, betas, mWs, mbs):
    del e
    f32 = jnp.float32

    h_pad = jnp.concatenate([h, jnp.zeros((NPAD - N,), jnp.int32)])
    hidx = h_pad.reshape(NW, 4, 80)
    npad_e = EPAD - E
    src_p = jnp.concatenate([edge_index[0], jnp.zeros((npad_e,), jnp.int32)])
    dst_p = jnp.concatenate(
        [edge_index[1], jnp.full((npad_e,), NPAD - 1, jnp.int32)])
    srcg = src_p.reshape(NW, NCH, K)
    dstg = dst_p.reshape(NW, NCH, K)
    ones_rows = jnp.ones((K, HID), f32)
    z128 = jnp.zeros((NPAD, HID), f32)

    def pad_mat(m, rows, cols):
        return jnp.zeros((rows, cols), f32).at[: m.shape[0], : m.shape[1]].set(m)

    m1 = pad_mat(mWs[0], HID, HID)
    m2 = pad_mat(mWs[1], HID, HID)
    m3 = pad_mat(mWs[2], HID, HID)
    c1 = pad_mat(mbs[0][None, :], 1, HID)
    c2 = pad_mat(mbs[1][None, :], 1, HID)
    c3 = pad_mat(mbs[2][None, :], 1, HID)

    sc_embed, sc_seg = _sc_kernels()
    x, cntp_wide = sc_embed(emb, hidx, dstg, ones_rows, z128)
    cntp = cntp_wide[:, :, :1]

    for l in range(L):
        parts = sc_seg(x, srcg, dstg, z128)
        b2 = bs[l][None, :]
        g2 = gammas[l][None, :]
        be2 = betas[l][None, :]
        if l < L - 1:
            x = _tc_layer(x, parts, cntp, Ws[l], b2, g2, be2)
        else:
            x = _tc_layer_mlp(x, parts, cntp, Ws[l], b2, g2, be2,
                              m1, c1, m2, c2, m3, c3)

    return x[:N, :NCLS]


# speedup vs baseline: 2.9711x; 1.0440x over previous
import functools

import jax
import jax.numpy as jnp
from jax import lax
from jax.experimental import pallas as pl
from jax.experimental.pallas import tpu as pltpu
from jax.experimental.pallas import tpu_sc as plsc

N = 10000
E = 320000
HID = 128
NCLS = 8
L = 4

NPAD = 10240
NW = 32
K = 128
NCH = 80
GSZ = 16
NG = NCH // GSZ
EPAD = NW * NCH * K
ROWS_PER_SUB = NPAD // 16


def _sc_embed_count_body(emb_hbm, hidx_hbm, dstg_hbm, ones_hbm, z128_hbm,
                    x0_hbm, cntp_hbm,
                    hidx_v, rows_v, dst_v, ones_v, cntacc, sem):
    c = lax.axis_index("c")
    s = lax.axis_index("s")
    w = c * 16 + s

    pltpu.sync_copy(hidx_hbm.at[w], hidx_v)
    for jj in range(4):
        pltpu.async_copy(emb_hbm.at[hidx_v.at[jj]], rows_v, sem).wait()
        pltpu.sync_copy(rows_v, x0_hbm.at[pl.ds(w * 320 + jj * 80, 80)])

    pltpu.sync_copy(z128_hbm.at[pl.ds(s * ROWS_PER_SUB, ROWS_PER_SUB)],
                    cntacc.at[pl.ds(s * ROWS_PER_SUB, ROWS_PER_SUB)])
    pltpu.sync_copy(ones_hbm, ones_v)
    plsc.subcore_barrier()

    for g in range(NG):
        pltpu.sync_copy(dstg_hbm.at[w, pl.ds(g * GSZ, GSZ)], dst_v)

        def body(j, _):
            pltpu.sync_copy(ones_v, cntacc.at[dst_v.at[j]], add=True)
            return ()
        lax.fori_loop(0, GSZ, body, ())

    plsc.subcore_barrier()
    pltpu.sync_copy(cntacc.at[pl.ds(s * ROWS_PER_SUB, ROWS_PER_SUB)],
                    cntp_hbm.at[c, pl.ds(s * ROWS_PER_SUB, ROWS_PER_SUB)])


def _sc_segment_sum_body(x_hbm, srcg_hbm, dstg_hbm, z128_hbm,
                    parts_hbm,
                    srcA, dstA, srcB, dstB, rows0, rows1, acc,
                    semG, semS, semI):
    c = lax.axis_index("c")
    s = lax.axis_index("s")
    w = c * 16 + s

    pltpu.sync_copy(z128_hbm.at[pl.ds(s * ROWS_PER_SUB, ROWS_PER_SUB)],
                    acc.at[pl.ds(s * ROWS_PER_SUB, ROWS_PER_SUB)])
    plsc.subcore_barrier()

    rows = (rows0, rows1)
    srcb = (srcA, srcB)
    dstb = (dstA, dstB)

    def wait_gather():
        pltpu.make_async_copy(x_hbm.at[srcA.at[0]], rows0, semG).wait()

    def wait_scatter():
        pltpu.make_async_copy(rows0, acc.at[dstA.at[0]], semS).wait()

    def wait_idx():
        pltpu.make_async_copy(srcg_hbm.at[w, pl.ds(0, GSZ)], srcA, semI).wait()

    pltpu.sync_copy(srcg_hbm.at[w, pl.ds(0, GSZ)], srcA)
    pltpu.sync_copy(dstg_hbm.at[w, pl.ds(0, GSZ)], dstA)
    for j in range(NCH):
        g, jj = divmod(j, GSZ)
        if jj == 1 and g + 1 < NG:
            nb = (g + 1) % 2
            pltpu.async_copy(
                srcg_hbm.at[w, pl.ds((g + 1) * GSZ, GSZ)], srcb[nb], semI)
            pltpu.async_copy(
                dstg_hbm.at[w, pl.ds((g + 1) * GSZ, GSZ)], dstb[nb], semI)
        if j >= 2:
            wait_scatter()
        pltpu.async_copy(x_hbm.at[srcb[g % 2].at[jj]], rows[j % 2], semG)
        if j >= 1:
            pg, pj = divmod(j - 1, GSZ)
            wait_gather()
            pltpu.async_copy(rows[(j - 1) % 2],
                             acc.at[dstb[pg % 2].at[pj]], semS, add=True)
        if jj == GSZ - 1 and g + 1 < NG:
            wait_idx()
            wait_idx()
    wait_gather()
    pltpu.async_copy(rows[(NCH - 1) % 2],
                     acc.at[dstb[(NG - 1) % 2].at[GSZ - 1]], semS, add=True)
    wait_scatter()
    wait_scatter()

    plsc.subcore_barrier()
    pltpu.sync_copy(acc.at[pl.ds(s * ROWS_PER_SUB, ROWS_PER_SUB)],
                    parts_hbm.at[c, pl.ds(s * ROWS_PER_SUB, ROWS_PER_SUB)])


@functools.cache
def _sc_kernels():
    mesh = plsc.VectorSubcoreMesh(core_axis_name="c", subcore_axis_name="s")
    embed = pl.kernel(
        _sc_embed_count_body,
        out_type=(
            jax.ShapeDtypeStruct((NPAD, HID), jnp.float32),
            jax.ShapeDtypeStruct((2, NPAD, HID), jnp.float32),
        ),
        mesh=mesh,
        scratch_types=[
            pltpu.VMEM((4, 80), jnp.int32),
            pltpu.VMEM((80, HID), jnp.float32),
            pltpu.VMEM((GSZ, K), jnp.int32),
            pltpu.VMEM((K, HID), jnp.float32),
            pltpu.VMEM_SHARED((NPAD, HID), jnp.float32),
            pltpu.SemaphoreType.DMA,
        ],
    )
    seg = pl.kernel(
        _sc_segment_sum_body,
        out_type=jax.ShapeDtypeStruct((2, NPAD, HID), jnp.float32),
        mesh=mesh,
        scratch_types=[
            pltpu.VMEM((GSZ, K), jnp.int32),
            pltpu.VMEM((GSZ, K), jnp.int32),
            pltpu.VMEM((GSZ, K), jnp.int32),
            pltpu.VMEM((GSZ, K), jnp.int32),
            pltpu.VMEM((K, HID), jnp.float32),
            pltpu.VMEM((K, HID), jnp.float32),
            pltpu.VMEM_SHARED((NPAD, HID), jnp.float32),
            pltpu.SemaphoreType.DMA,
            pltpu.SemaphoreType.DMA,
            pltpu.SemaphoreType.DMA,
        ],
    )
    return embed, seg


def _tc_layer_body(x_ref, parts_ref, cntp_ref, w_ref, b_ref, g_ref, be_ref,
                   o_ref, *, mlp_refs=None):
    x = x_ref[...]
    agg = parts_ref[0] + parts_ref[1]
    cnt = cntp_ref[0] + cntp_ref[1]
    neigh = agg / jnp.maximum(cnt, 1.0)
    w_self = w_ref[:HID, :]
    w_neigh = w_ref[HID:, :]
    bundle = (
        jnp.dot(x, w_self, preferred_element_type=jnp.float32)
        + jnp.dot(neigh, w_neigh, preferred_element_type=jnp.float32)
        + b_ref[...]
    )
    ss = jnp.sum(bundle * bundle, axis=1, keepdims=True)
    bundle = bundle / jnp.maximum(jnp.sqrt(ss), 1e-12)
    bundle = jnp.maximum(bundle, 0.0)
    real = bundle[:N, :]
    mean = jnp.sum(real, axis=0, keepdims=True) * (1.0 / N)
    var = jnp.sum((real - mean) * (real - mean), axis=0, keepdims=True) * (1.0 / N)
    bundle = (bundle - mean) * lax.rsqrt(var + 1e-5) * g_ref[...] + be_ref[...]
    out = x + bundle
    if mlp_refs is None:
        o_ref[...] = out
    else:
        (m1, c1, m2, c2, m3, c3) = mlp_refs
        y = jnp.dot(out, m1[...], preferred_element_type=jnp.float32) + c1[...]
        y = jnp.maximum(y, 0.0)
        y = jnp.dot(y, m2[...], preferred_element_type=jnp.float32) + c2[...]
        y = jnp.maximum(y, 0.0)
        y = jnp.dot(y, m3[...], preferred_element_type=jnp.float32) + c3[...]
        o_ref[...] = y


def _tc_layer(x, parts, cntp, w, b, g, be):
    return pl.pallas_call(
        _tc_layer_body,
        out_shape=jax.ShapeDtypeStruct((NPAD, HID), jnp.float32),
    )(x, parts, cntp, w, b, g, be)


def _tc_layer_mlp(x, parts, cntp, w, b, g, be, m1, c1, m2, c2, m3, c3):
    def body(x_ref, parts_ref, cntp_ref, w_ref, b_ref, g_ref, be_ref,
             m1r, c1r, m2r, c2r, m3r, c3r, o_ref):
        _tc_layer_body(x_ref, parts_ref, cntp_ref, w_ref, b_ref, g_ref,
                       be_ref, o_ref,
                       mlp_refs=(m1r, c1r, m2r, c2r, m3r, c3r))
    return pl.pallas_call(
        body,
        out_shape=jax.ShapeDtypeStruct((NPAD, HID), jnp.float32),
    )(x, parts, cntp, w, b, g, be, m1, c1, m2, c2, m3, c3)


def kernel(h, edge_index, e, emb, Ws, bs, gammas, betas, mWs, mbs):
    del e
    f32 = jnp.float32

    h_pad = jnp.concatenate([h, jnp.zeros((NPAD - N,), jnp.int32)])
    hidx = h_pad.reshape(NW, 4, 80)
    npad_e = EPAD - E
    src_p = jnp.concatenate([edge_index[0], jnp.zeros((npad_e,), jnp.int32)])
    dst_p = jnp.concatenate(
        [edge_index[1], jnp.full((npad_e,), NPAD - 1, jnp.int32)])
    srcg = src_p.reshape(NW, NCH, K)
    dstg = dst_p.reshape(NW, NCH, K)
    ones_rows = jnp.ones((K, HID), f32)
    z128 = jnp.zeros((NPAD, HID), f32)

    def pad_mat(m, rows, cols):
        return jnp.zeros((rows, cols), f32).at[: m.shape[0], : m.shape[1]].set(m)

    m1 = pad_mat(mWs[0], HID, HID)
    m2 = pad_mat(mWs[1], HID, HID)
    m3 = pad_mat(mWs[2], HID, HID)
    c1 = pad_mat(mbs[0][None, :], 1, HID)
    c2 = pad_mat(mbs[1][None, :], 1, HID)
    c3 = pad_mat(mbs[2][None, :], 1, HID)

    sc_embed, sc_seg = _sc_kernels()
    x, cntp_wide = sc_embed(emb, hidx, dstg, ones_rows, z128)
    cntp = cntp_wide[:, :, :1]

    for l in range(L):
        parts = sc_seg(x, srcg, dstg, z128)
        b2 = bs[l][None, :]
        g2 = gammas[l][None, :]
        be2 = betas[l][None, :]
        if l < L - 1:
            x = _tc_layer(x, parts, cntp, Ws[l], b2, g2, be2)
        else:
            x = _tc_layer_mlp(x, parts, cntp, Ws[l], b2, g2, be2,
                              m1, c1, m2, c2, m3, c3)

    return x[:N, :NCLS]


# speedup vs baseline: 3.5454x; 1.1933x over previous
import functools

import jax
import jax.numpy as jnp
from jax import lax
from jax.experimental import pallas as pl
from jax.experimental.pallas import tpu as pltpu
from jax.experimental.pallas import tpu_sc as plsc

N = 10000
E = 320000
HID = 128
NCLS = 8
L = 4

NPAD = 10240
NW = 32
K = 128
NCH = 80
GSZ = 16
NG = NCH // GSZ
EPAD = NW * NCH * K
ROWS_PER_SUB = NPAD // 16


def _sc_embed_count_body(emb_hbm, hidx_hbm, dstg_hbm, ones_hbm, z128_hbm,
                    x0_hbm, cntp_hbm,
                    hidx_v, rows_v, dst_v, ones_v, cntacc, sem):
    c = lax.axis_index("c")
    s = lax.axis_index("s")
    w = c * 16 + s

    pltpu.sync_copy(hidx_hbm.at[w], hidx_v)
    for jj in range(4):
        pltpu.async_copy(emb_hbm.at[hidx_v.at[jj]], rows_v, sem).wait()
        pltpu.sync_copy(rows_v, x0_hbm.at[pl.ds(w * 320 + jj * 80, 80)])

    pltpu.sync_copy(z128_hbm.at[pl.ds(s * ROWS_PER_SUB, ROWS_PER_SUB)],
                    cntacc.at[pl.ds(s * ROWS_PER_SUB, ROWS_PER_SUB)])
    pltpu.sync_copy(ones_hbm, ones_v)
    plsc.subcore_barrier()

    for g in range(NG):
        pltpu.sync_copy(dstg_hbm.at[w, pl.ds(g * GSZ, GSZ)], dst_v)

        def body(j, _):
            pltpu.sync_copy(ones_v, cntacc.at[dst_v.at[j]], add=True)
            return ()
        lax.fori_loop(0, GSZ, body, ())

    plsc.subcore_barrier()
    pltpu.sync_copy(cntacc.at[pl.ds(s * ROWS_PER_SUB, ROWS_PER_SUB)],
                    cntp_hbm.at[c, pl.ds(s * ROWS_PER_SUB, ROWS_PER_SUB)])


def _sc_segment_sum_body(x_hbm, srcg_hbm, dstg_hbm, z128_hbm,
                    parts_hbm,
                    srcA, dstA, srcB, dstB, rows0, rows1, acc,
                    semG, semS, semI):
    c = lax.axis_index("c")
    s = lax.axis_index("s")
    w = c * 16 + s

    pltpu.sync_copy(z128_hbm.at[pl.ds(s * ROWS_PER_SUB, ROWS_PER_SUB)],
                    acc.at[pl.ds(s * ROWS_PER_SUB, ROWS_PER_SUB)])
    plsc.subcore_barrier()

    rows = (rows0, rows1)
    srcb = (srcA, srcB)
    dstb = (dstA, dstB)

    def wait_gather():
        pltpu.make_async_copy(x_hbm.at[srcA.at[0]], rows0, semG).wait()

    def wait_scatter():
        pltpu.make_async_copy(rows0, acc.at[dstA.at[0]], semS).wait()

    def wait_idx():
        pltpu.make_async_copy(srcg_hbm.at[w, pl.ds(0, GSZ)], srcA, semI).wait()

    pltpu.sync_copy(srcg_hbm.at[w, pl.ds(0, GSZ)], srcA)
    pltpu.sync_copy(dstg_hbm.at[w, pl.ds(0, GSZ)], dstA)
    for j in range(NCH):
        g, jj = divmod(j, GSZ)
        if jj == 1 and g + 1 < NG:
            nb = (g + 1) % 2
            pltpu.async_copy(
                srcg_hbm.at[w, pl.ds((g + 1) * GSZ, GSZ)], srcb[nb], semI)
            pltpu.async_copy(
                dstg_hbm.at[w, pl.ds((g + 1) * GSZ, GSZ)], dstb[nb], semI)
        if j >= 2:
            wait_scatter()
        pltpu.async_copy(x_hbm.at[srcb[g % 2].at[jj]], rows[j % 2], semG)
        if j >= 1:
            pg, pj = divmod(j - 1, GSZ)
            wait_gather()
            pltpu.async_copy(rows[(j - 1) % 2],
                             acc.at[dstb[pg % 2].at[pj]], semS, add=True)
        if jj == GSZ - 1 and g + 1 < NG:
            wait_idx()
            wait_idx()
    wait_gather()
    pltpu.async_copy(rows[(NCH - 1) % 2],
                     acc.at[dstb[(NG - 1) % 2].at[GSZ - 1]], semS, add=True)
    wait_scatter()
    wait_scatter()

    plsc.subcore_barrier()
    pltpu.sync_copy(acc.at[pl.ds(s * ROWS_PER_SUB, ROWS_PER_SUB)],
                    parts_hbm.at[c, pl.ds(s * ROWS_PER_SUB, ROWS_PER_SUB)])


@functools.cache
def _sc_kernels():
    mesh = plsc.VectorSubcoreMesh(core_axis_name="c", subcore_axis_name="s")
    embed = pl.kernel(
        _sc_embed_count_body,
        out_type=(
            jax.ShapeDtypeStruct((NPAD, HID), jnp.float32),
            jax.ShapeDtypeStruct((2, NPAD, HID), jnp.float32),
        ),
        mesh=mesh,
        scratch_types=[
            pltpu.VMEM((4, 80), jnp.int32),
            pltpu.VMEM((80, HID), jnp.float32),
            pltpu.VMEM((GSZ, K), jnp.int32),
            pltpu.VMEM((K, HID), jnp.float32),
            pltpu.VMEM_SHARED((NPAD, HID), jnp.float32),
            pltpu.SemaphoreType.DMA,
        ],
    )
    seg = pl.kernel(
        _sc_segment_sum_body,
        out_type=jax.ShapeDtypeStruct((2, NPAD, HID), jnp.float32),
        mesh=mesh,
        scratch_types=[
            pltpu.VMEM((GSZ, K), jnp.int32),
            pltpu.VMEM((GSZ, K), jnp.int32),
            pltpu.VMEM((GSZ, K), jnp.int32),
            pltpu.VMEM((GSZ, K), jnp.int32),
            pltpu.VMEM((K, HID), jnp.float32),
            pltpu.VMEM((K, HID), jnp.float32),
            pltpu.VMEM_SHARED((NPAD, HID), jnp.float32),
            pltpu.SemaphoreType.DMA,
            pltpu.SemaphoreType.DMA,
            pltpu.SemaphoreType.DMA,
        ],
    )
    return embed, seg


def _tc_layer_body(x_ref, parts_ref, cntp_ref, w_ref, b_ref, g_ref, be_ref,
                   o_ref, *, mlp_refs=None):
    x = x_ref[...]
    agg = parts_ref[0] + parts_ref[1]
    cnt = cntp_ref[0] + cntp_ref[1]
    neigh = agg / jnp.maximum(cnt, 1.0)
    w_self = w_ref[:HID, :]
    w_neigh = w_ref[HID:, :]
    bundle = (
        jnp.dot(x, w_self, preferred_element_type=jnp.float32)
        + jnp.dot(neigh, w_neigh, preferred_element_type=jnp.float32)
        + b_ref[...]
    )
    ss = jnp.sum(bundle * bundle, axis=1, keepdims=True)
    bundle = bundle / jnp.maximum(jnp.sqrt(ss), 1e-12)
    bundle = jnp.maximum(bundle, 0.0)
    real = bundle[:N, :]
    mean = jnp.sum(real, axis=0, keepdims=True) * (1.0 / N)
    var = jnp.sum((real - mean) * (real - mean), axis=0, keepdims=True) * (1.0 / N)
    bundle = (bundle - mean) * lax.rsqrt(var + 1e-5) * g_ref[...] + be_ref[...]
    out = x + bundle
    if mlp_refs is None:
        o_ref[...] = out
    else:
        (m1, c1, m2, c2, m3, c3) = mlp_refs
        y = jnp.dot(out, m1[...], preferred_element_type=jnp.float32) + c1[...]
        y = jnp.maximum(y, 0.0)
        y = jnp.dot(y, m2[...], preferred_element_type=jnp.float32) + c2[...]
        y = jnp.maximum(y, 0.0)
        y = jnp.dot(y, m3[...], preferred_element_type=jnp.float32) + c3[...]
        o_ref[...] = y


def _tc_layer(x, parts, cntp, w, b, g, be):
    return pl.pallas_call(
        _tc_layer_body,
        out_shape=jax.ShapeDtypeStruct((NPAD, HID), jnp.float32),
    )(x, parts, cntp, w, b, g, be)


def _tc_layer_mlp(x, parts, cntp, w, b, g, be, m1, c1, m2, c2, m3, c3):
    def body(x_ref, parts_ref, cntp_ref, w_ref, b_ref, g_ref, be_ref,
             m1r, c1r, m2r, c2r, m3r, c3r, o_ref):
        _tc_layer_body(x_ref, parts_ref, cntp_ref, w_ref, b_ref, g_ref,
                       be_ref, o_ref,
                       mlp_refs=(m1r, c1r, m2r, c2r, m3r, c3r))
    return pl.pallas_call(
        body,
        out_shape=jax.ShapeDtypeStruct((NPAD, HID), jnp.float32),
    )(x, parts, cntp, w, b, g, be, m1, c1, m2, c2, m3, c3)


def kernel(h, edge_index, e, emb, Ws, bs, gammas, betas, mWs, mbs):
    del e
    f32 = jnp.float32

    h_pad = jnp.concatenate([h, jnp.zeros((NPAD - N,), jnp.int32)])
    hidx = h_pad.reshape(NW, 4, 80)
    npad_e = EPAD - E
    src_p = jnp.concatenate([edge_index[0], jnp.zeros((npad_e,), jnp.int32)])
    dst_p = jnp.concatenate(
        [edge_index[1],
         N + (jnp.arange(npad_e, dtype=jnp.int32) % (NPAD - N))])
    srcg = src_p.reshape(NCH, NW, K).transpose(1, 0, 2)
    dstg = dst_p.reshape(NCH, NW, K).transpose(1, 0, 2)
    ones_rows = jnp.ones((K, HID), f32)
    z128 = jnp.zeros((NPAD, HID), f32)

    def pad_mat(m, rows, cols):
        return jnp.zeros((rows, cols), f32).at[: m.shape[0], : m.shape[1]].set(m)

    m1 = pad_mat(mWs[0], HID, HID)
    m2 = pad_mat(mWs[1], HID, HID)
    m3 = pad_mat(mWs[2], HID, HID)
    c1 = pad_mat(mbs[0][None, :], 1, HID)
    c2 = pad_mat(mbs[1][None, :], 1, HID)
    c3 = pad_mat(mbs[2][None, :], 1, HID)

    sc_embed, sc_seg = _sc_kernels()
    x, cntp_wide = sc_embed(emb, hidx, dstg, ones_rows, z128)
    cntp = cntp_wide[:, :, :1]

    for l in range(L):
        parts = sc_seg(x, srcg, dstg, z128)
        b2 = bs[l][None, :]
        g2 = gammas[l][None, :]
        be2 = betas[l][None, :]
        if l < L - 1:
            x = _tc_layer(x, parts, cntp, Ws[l], b2, g2, be2)
        else:
            x = _tc_layer_mlp(x, parts, cntp, Ws[l], b2, g2, be2,
                              m1, c1, m2, c2, m3, c3)

    return x[:N, :NCLS]


# speedup vs baseline: 3.5749x; 1.0083x over previous
import functools

import jax
import jax.numpy as jnp
from jax import lax
from jax.experimental import pallas as pl
from jax.experimental.pallas import tpu as pltpu
from jax.experimental.pallas import tpu_sc as plsc

N = 10000
E = 320000
HID = 128
NCLS = 8
L = 4
EVOC = 128

NPAD = 10240
NW = 32
K = 128
NCH = 80
GSZ = 16
NG = NCH // GSZ
EPAD = NW * NCH * K
ROWS_PER_SUB = NPAD // 16


def _sc_embed_count_body(emb_hbm, hidx_hbm, dstg_hbm, ones_hbm, z128_hbm,
                    x0_hbm, cntp_hbm,
                    hidx_v, rows_v, dst_v, ones_v, embres, cntacc, sem):
    c = lax.axis_index("c")
    s = lax.axis_index("s")
    w = c * 16 + s

    pltpu.sync_copy(emb_hbm.at[pl.ds(s * 8, 8)], embres.at[pl.ds(s * 8, 8)])
    pltpu.sync_copy(z128_hbm.at[pl.ds(s * ROWS_PER_SUB, ROWS_PER_SUB)],
                    cntacc.at[pl.ds(s * ROWS_PER_SUB, ROWS_PER_SUB)])
    pltpu.sync_copy(ones_hbm, ones_v)
    plsc.subcore_barrier()

    pltpu.sync_copy(hidx_hbm.at[w], hidx_v)
    for jj in range(4):
        pltpu.async_copy(embres.at[hidx_v.at[jj]], rows_v, sem).wait()
        pltpu.sync_copy(rows_v, x0_hbm.at[pl.ds(w * 320 + jj * 80, 80)])

    for g in range(NG):
        pltpu.sync_copy(dstg_hbm.at[w, pl.ds(g * GSZ, GSZ)], dst_v)

        def body(j, _):
            pltpu.sync_copy(ones_v, cntacc.at[dst_v.at[j]], add=True)
            return ()
        lax.fori_loop(0, GSZ, body, ())

    plsc.subcore_barrier()
    pltpu.sync_copy(cntacc.at[pl.ds(s * ROWS_PER_SUB, ROWS_PER_SUB)],
                    cntp_hbm.at[c, pl.ds(s * ROWS_PER_SUB, ROWS_PER_SUB)])


def _sc_segment_sum_body(x_hbm, srcg_hbm, dstg_hbm, z128_hbm,
                    parts_hbm,
                    srcA, dstA, srcB, dstB, rows0, rows1, acc,
                    semG, semS, semI):
    c = lax.axis_index("c")
    s = lax.axis_index("s")
    w = c * 16 + s

    pltpu.sync_copy(z128_hbm.at[pl.ds(s * ROWS_PER_SUB, ROWS_PER_SUB)],
                    acc.at[pl.ds(s * ROWS_PER_SUB, ROWS_PER_SUB)])
    plsc.subcore_barrier()

    rows = (rows0, rows1)
    srcb = (srcA, srcB)
    dstb = (dstA, dstB)

    def wait_gather():
        pltpu.make_async_copy(x_hbm.at[srcA.at[0]], rows0, semG).wait()

    def wait_scatter():
        pltpu.make_async_copy(rows0, acc.at[dstA.at[0]], semS).wait()

    def wait_idx():
        pltpu.make_async_copy(srcg_hbm.at[w, pl.ds(0, GSZ)], srcA, semI).wait()

    pltpu.sync_copy(srcg_hbm.at[w, pl.ds(0, GSZ)], srcA)
    pltpu.sync_copy(dstg_hbm.at[w, pl.ds(0, GSZ)], dstA)
    for j in range(NCH):
        g, jj = divmod(j, GSZ)
        if jj == 1 and g + 1 < NG:
            nb = (g + 1) % 2
            pltpu.async_copy(
                srcg_hbm.at[w, pl.ds((g + 1) * GSZ, GSZ)], srcb[nb], semI)
            pltpu.async_copy(
                dstg_hbm.at[w, pl.ds((g + 1) * GSZ, GSZ)], dstb[nb], semI)
        if j >= 2:
            wait_scatter()
        pltpu.async_copy(x_hbm.at[srcb[g % 2].at[jj]], rows[j % 2], semG)
        if j >= 1:
            pg, pj = divmod(j - 1, GSZ)
            wait_gather()
            pltpu.async_copy(rows[(j - 1) % 2],
                             acc.at[dstb[pg % 2].at[pj]], semS, add=True)
        if jj == GSZ - 1 and g + 1 < NG:
            wait_idx()
            wait_idx()
    wait_gather()
    pltpu.async_copy(rows[(NCH - 1) % 2],
                     acc.at[dstb[(NG - 1) % 2].at[GSZ - 1]], semS, add=True)
    wait_scatter()
    wait_scatter()

    plsc.subcore_barrier()
    pltpu.sync_copy(acc.at[pl.ds(s * ROWS_PER_SUB, ROWS_PER_SUB)],
                    parts_hbm.at[c, pl.ds(s * ROWS_PER_SUB, ROWS_PER_SUB)])


@functools.cache
def _sc_kernels():
    mesh = plsc.VectorSubcoreMesh(core_axis_name="c", subcore_axis_name="s")
    embed = pl.kernel(
        _sc_embed_count_body,
        out_type=(
            jax.ShapeDtypeStruct((NPAD, HID), jnp.float32),
            jax.ShapeDtypeStruct((2, NPAD, HID), jnp.float32),
        ),
        mesh=mesh,
        scratch_types=[
            pltpu.VMEM((4, 80), jnp.int32),
            pltpu.VMEM((80, HID), jnp.float32),
            pltpu.VMEM((GSZ, K), jnp.int32),
            pltpu.VMEM((K, HID), jnp.float32),
            pltpu.VMEM_SHARED((EVOC, HID), jnp.float32),
            pltpu.VMEM_SHARED((NPAD, HID), jnp.float32),
            pltpu.SemaphoreType.DMA,
        ],
    )
    seg = pl.kernel(
        _sc_segment_sum_body,
        out_type=jax.ShapeDtypeStruct((2, NPAD, HID), jnp.float32),
        mesh=mesh,
        scratch_types=[
            pltpu.VMEM((GSZ, K), jnp.int32),
            pltpu.VMEM((GSZ, K), jnp.int32),
            pltpu.VMEM((GSZ, K), jnp.int32),
            pltpu.VMEM((GSZ, K), jnp.int32),
            pltpu.VMEM((K, HID), jnp.float32),
            pltpu.VMEM((K, HID), jnp.float32),
            pltpu.VMEM_SHARED((NPAD, HID), jnp.float32),
            pltpu.SemaphoreType.DMA,
            pltpu.SemaphoreType.DMA,
            pltpu.SemaphoreType.DMA,
        ],
    )
    return embed, seg


def _tc_layer_body(x_ref, parts_ref, cntp_ref, w_ref, b_ref, g_ref, be_ref,
                   o_ref, *, mlp_refs=None):
    x = x_ref[...]
    agg = parts_ref[0] + parts_ref[1]
    cnt = cntp_ref[0] + cntp_ref[1]
    neigh = agg / jnp.maximum(cnt, 1.0)
    w_self = w_ref[:HID, :]
    w_neigh = w_ref[HID:, :]
    bundle = (
        jnp.dot(x, w_self, preferred_element_type=jnp.float32)
        + jnp.dot(neigh, w_neigh, preferred_element_type=jnp.float32)
        + b_ref[...]
    )
    ss = jnp.sum(bundle * bundle, axis=1, keepdims=True)
    bundle = bundle / jnp.maximum(jnp.sqrt(ss), 1e-12)
    bundle = jnp.maximum(bundle, 0.0)
    real = bundle[:N, :]
    mean = jnp.sum(real, axis=0, keepdims=True) * (1.0 / N)
    var = jnp.sum((real - mean) * (real - mean), axis=0, keepdims=True) * (1.0 / N)
    bundle = (bundle - mean) * lax.rsqrt(var + 1e-5) * g_ref[...] + be_ref[...]
    out = x + bundle
    if mlp_refs is None:
        o_ref[...] = out
    else:
        (m1, c1, m2, c2, m3, c3) = mlp_refs
        y = jnp.dot(out, m1[...], preferred_element_type=jnp.float32) + c1[...]
        y = jnp.maximum(y, 0.0)
        y = jnp.dot(y, m2[...], preferred_element_type=jnp.float32) + c2[...]
        y = jnp.maximum(y, 0.0)
        y = jnp.dot(y, m3[...], preferred_element_type=jnp.float32) + c3[...]
        o_ref[...] = y


def _tc_layer(x, parts, cntp, w, b, g, be):
    return pl.pallas_call(
        _tc_layer_body,
        out_shape=jax.ShapeDtypeStruct((NPAD, HID), jnp.float32),
    )(x, parts, cntp, w, b, g, be)


def _tc_layer_mlp(x, parts, cntp, w, b, g, be, m1, c1, m2, c2, m3, c3):
    def body(x_ref, parts_ref, cntp_ref, w_ref, b_ref, g_ref, be_ref,
             m1r, c1r, m2r, c2r, m3r, c3r, o_ref):
        _tc_layer_body(x_ref, parts_ref, cntp_ref, w_ref, b_ref, g_ref,
                       be_ref, o_ref,
                       mlp_refs=(m1r, c1r, m2r, c2r, m3r, c3r))
    return pl.pallas_call(
        body,
        out_shape=jax.ShapeDtypeStruct((NPAD, HID), jnp.float32),
    )(x, parts, cntp, w, b, g, be, m1, c1, m2, c2, m3, c3)


def kernel(h, edge_index, e, emb, Ws, bs, gammas, betas, mWs, mbs):
    del e
    f32 = jnp.float32

    h_pad = jnp.concatenate([h, jnp.zeros((NPAD - N,), jnp.int32)])
    hidx = h_pad.reshape(NW, 4, 80)
    npad_e = EPAD - E
    src_p = jnp.concatenate([edge_index[0], jnp.zeros((npad_e,), jnp.int32)])
    dst_p = jnp.concatenate(
        [edge_index[1],
         N + (jnp.arange(npad_e, dtype=jnp.int32) % (NPAD - N))])
    srcg = src_p.reshape(NCH, NW, K).transpose(1, 0, 2)
    dstg = dst_p.reshape(NCH, NW, K).transpose(1, 0, 2)
    ones_rows = jnp.ones((K, HID), f32)
    z128 = jnp.zeros((NPAD, HID), f32)

    def pad_mat(m, rows, cols):
        return jnp.zeros((rows, cols), f32).at[: m.shape[0], : m.shape[1]].set(m)

    m1 = pad_mat(mWs[0], HID, HID)
    m2 = pad_mat(mWs[1], HID, HID)
    m3 = pad_mat(mWs[2], HID, HID)
    c1 = pad_mat(mbs[0][None, :], 1, HID)
    c2 = pad_mat(mbs[1][None, :], 1, HID)
    c3 = pad_mat(mbs[2][None, :], 1, HID)

    sc_embed, sc_seg = _sc_kernels()
    x, cntp_wide = sc_embed(emb, hidx, dstg, ones_rows, z128)
    cntp = cntp_wide[:, :, :1]

    for l in range(L):
        parts = sc_seg(x, srcg, dstg, z128)
        b2 = bs[l][None, :]
        g2 = gammas[l][None, :]
        be2 = betas[l][None, :]
        if l < L - 1:
            x = _tc_layer(x, parts, cntp, Ws[l], b2, g2, be2)
        else:
            x = _tc_layer_mlp(x, parts, cntp, Ws[l], b2, g2, be2,
                              m1, c1, m2, c2, m3, c3)

    return x[:N, :NCLS]
